# Initial kernel scaffold; baseline (speedup 1.0000x reference)
import jax
import jax.numpy as jnp
from jax.experimental import pallas as pl


def kernel(feat, edge_index, W_self, b_self, W_neigh, b_neigh):
    raise NotImplementedError("write your pallas kernel here")


# speedup vs baseline: 2.4264x; 2.4264x over previous
import functools

import jax
import jax.numpy as jnp
from jax import lax
from jax.experimental import pallas as pl
from jax.experimental.pallas import tpu as pltpu
from jax.experimental.pallas import tpu_sc as plsc

N = 10000
E = 160000
D = 256
DQ = 64
NC = 2
NS = 16
L = 16
EPT = E // NS
CH = 80
NCH = EPT // CH
NP = 10240
NRT = NP // NS
DW = 16
SS = NRT // CH


def _sc_body(tbl, srcs, dsts, zeros, zeros_d,
             out_q0, out_q1, out_q2, out_q3, out_deg,
             src_v, idx_v, dst_v, rows_v, ones_v, dstage_v, acc_s, deg_s, sem):
    c = lax.axis_index("c")
    s = lax.axis_index("s")
    row0 = s * NRT
    base0 = s * EPT

    ones = jnp.full((L,), 1.0, dtype=jnp.float32)
    for r in range(CH):
        ones_v[r, :] = ones
    pltpu.sync_copy(zeros_d.at[pl.ds(row0, NRT)], dstage_v)
    pltpu.sync_copy(dstage_v, deg_s.at[pl.ds(row0, NRT)])

    for p in range(2):
        pltpu.sync_copy(zeros.at[pl.ds(0, CH)], rows_v)
        for k in range(SS):
            pltpu.sync_copy(rows_v, acc_s.at[pl.ds(row0 + k * CH, CH)])
        plsc.subcore_barrier()

        off = (2 * c + p) * N

        def chunk(i, carry, do_deg=(p == 0)):
            eb = base0 + i * CH
            pltpu.sync_copy(srcs.at[pl.ds(eb, CH)], src_v)
            pltpu.sync_copy(dsts.at[pl.ds(eb, CH)], dst_v)
            for j in range(CH // L):
                idx_v[pl.ds(j * L, L)] = src_v[pl.ds(j * L, L)] + off
            pltpu.async_copy(tbl.at[idx_v], rows_v, sem).wait()
            pltpu.sync_copy(rows_v, acc_s.at[dst_v], add=True)
            if do_deg:
                pltpu.sync_copy(ones_v, deg_s.at[dst_v], add=True)
            return carry

        lax.fori_loop(0, NCH, chunk, 0)
        plsc.subcore_barrier()

        for k in range(SS):
            pltpu.sync_copy(acc_s.at[pl.ds(row0 + k * CH, CH)], rows_v)
            dst_lo = (out_q0, out_q1)[p]
            dst_hi = (out_q2, out_q3)[p]

            @pl.when(c == 0)
            def _(dst=dst_lo, k=k):
                pltpu.sync_copy(rows_v, dst.at[pl.ds(row0 + k * CH, CH)])

            @pl.when(c == 1)
            def _(dst=dst_hi, k=k):
                pltpu.sync_copy(rows_v, dst.at[pl.ds(row0 + k * CH, CH)])

        if p == 0:
            @pl.when(c == 0)
            def _():
                pltpu.sync_copy(deg_s.at[pl.ds(row0, NRT)], dstage_v)
                pltpu.sync_copy(dstage_v, out_deg.at[pl.ds(row0, NRT)])


_sc_agg = functools.partial(
    pl.kernel,
    out_type=[
        jax.ShapeDtypeStruct((NP, DQ), jnp.float32),
        jax.ShapeDtypeStruct((NP, DQ), jnp.float32),
        jax.ShapeDtypeStruct((NP, DQ), jnp.float32),
        jax.ShapeDtypeStruct((NP, DQ), jnp.float32),
        jax.ShapeDtypeStruct((NP, DW), jnp.float32),
    ],
    mesh=plsc.VectorSubcoreMesh(core_axis_name="c", subcore_axis_name="s"),
    compiler_params=pltpu.CompilerParams(use_tc_tiling_on_sc=False),
    scratch_types=[
        pltpu.VMEM((CH,), jnp.int32),
        pltpu.VMEM((CH,), jnp.int32),
        pltpu.VMEM((CH,), jnp.int32),
        pltpu.VMEM((CH, DQ), jnp.float32),
        pltpu.VMEM((CH, DW), jnp.float32),
        pltpu.VMEM((NRT, DW), jnp.float32),
        pltpu.VMEM_SHARED((NP, DQ), jnp.float32),
        pltpu.VMEM_SHARED((NP, DW), jnp.float32),
        pltpu.SemaphoreType.DMA,
    ],
)(_sc_body)


BR = 1000


def _tc_body(feat_b, n0_b, n1_b, n2_b, n3_b, deg_b,
             ws, wn0, wn1, wn2, wn3, bias, out_b):
    inv = 1.0 / jnp.maximum(deg_b[...], 1.0)
    acc = jnp.dot(feat_b[...], ws[...], preferred_element_type=jnp.float32)
    acc += jnp.dot(n0_b[...] * inv, wn0[...], preferred_element_type=jnp.float32)
    acc += jnp.dot(n1_b[...] * inv, wn1[...], preferred_element_type=jnp.float32)
    acc += jnp.dot(n2_b[...] * inv, wn2[...], preferred_element_type=jnp.float32)
    acc += jnp.dot(n3_b[...] * inv, wn3[...], preferred_element_type=jnp.float32)
    out_b[...] = acc + bias[...]


def kernel(feat, edge_index, W_self, b_self, W_neigh, b_neigh):
    feat = feat.astype(jnp.float32)
    srcs = edge_index[0].astype(jnp.int32)
    dsts = edge_index[1].astype(jnp.int32)
    tbl = jnp.concatenate(
        [feat[:, 0 * DQ:1 * DQ], feat[:, 1 * DQ:2 * DQ],
         feat[:, 2 * DQ:3 * DQ], feat[:, 3 * DQ:4 * DQ]], axis=0)
    zeros = jnp.zeros((NP, DQ), dtype=jnp.float32)
    zeros_d = jnp.zeros((NP, DW), dtype=jnp.float32)

    n0, n1, n2, n3, deg16 = _sc_agg(tbl, srcs, dsts, zeros, zeros_d)
    deg = deg16[:N, :1]

    nblk = N // BR
    qspec = pl.BlockSpec((BR, DQ), lambda i: (i, 0))
    wspec = pl.BlockSpec((DQ, D), lambda i: (0, 0))
    out = pl.pallas_call(
        _tc_body,
        grid=(nblk,),
        in_specs=[
            pl.BlockSpec((BR, D), lambda i: (i, 0)),
            qspec, qspec, qspec, qspec,
            pl.BlockSpec((BR, 1), lambda i: (i, 0)),
            pl.BlockSpec((D, D), lambda i: (0, 0)),
            wspec, wspec, wspec, wspec,
            pl.BlockSpec((1, D), lambda i: (0, 0)),
        ],
        out_specs=pl.BlockSpec((BR, D), lambda i: (i, 0)),
        out_shape=jax.ShapeDtypeStruct((N, D), jnp.float32),
    )(feat, n0, n1, n2, n3, deg,
      W_self, W_neigh[0 * DQ:1 * DQ], W_neigh[1 * DQ:2 * DQ],
      W_neigh[2 * DQ:3 * DQ], W_neigh[3 * DQ:4 * DQ],
      (b_self + b_neigh).reshape(1, D))
    return out


# speedup vs baseline: 4.3213x; 1.7809x over previous
import functools

import jax
import jax.numpy as jnp
from jax import lax
from jax.experimental import pallas as pl
from jax.experimental.pallas import tpu as pltpu
from jax.experimental.pallas import tpu_sc as plsc

N = 10000
E = 160000
D = 256
DQ = 64
NC = 2
NS = 16
L = 16
EPT = E // NS
CH = 80
NCH = EPT // CH
NP = 10240
NRT = NP // NS
DW = 16
SS = NRT // CH
NROWS = E // CH


def _sc_body(tbl, sq0, sq1, sq2, sq3, d2d, zeros, zeros_d,
             out_q0, out_q1, out_q2, out_q3, out_deg,
             idx_all, dst_all, rows0, rows1, ones_v, dstage_v, acc_s, deg_s,
             sg0, sg1, ss0, ss1, sd0, sd1):
    c = lax.axis_index("c")
    s = lax.axis_index("s")
    row0 = s * NRT
    slab0 = s * NCH

    ones = jnp.full((L,), 1.0, dtype=jnp.float32)
    for r in range(CH):
        ones_v[r, :] = ones
    pltpu.sync_copy(d2d.at[pl.ds(slab0, NCH)], dst_all)
    pltpu.sync_copy(zeros_d, dstage_v)
    pltpu.sync_copy(dstage_v, deg_s.at[pl.ds(row0, NRT)])

    def gather_start(j, rows, sem):
        pltpu.async_copy(tbl.at[idx_all.at[j]], rows, sem)

    def gather_wait(rows, sem):
        pltpu.make_async_copy(tbl.at[idx_all.at[0]], rows, sem).wait()

    def scat_start(j, rows, ssem, dsem, do_deg):
        pltpu.async_copy(rows, acc_s.at[dst_all.at[j]], ssem, add=True)
        if do_deg:
            pltpu.async_copy(ones_v, deg_s.at[dst_all.at[j]], dsem, add=True)

    def scat_wait(rows, ssem, dsem, do_deg):
        pltpu.make_async_copy(rows, acc_s.at[dst_all.at[0]], ssem).wait()
        if do_deg:
            pltpu.make_async_copy(ones_v, deg_s.at[dst_all.at[0]], dsem).wait()

    for p in range(2):
        do_deg = p == 0

        src_lo = (sq0, sq1)[p]
        src_hi = (sq2, sq3)[p]

        @pl.when(c == 0)
        def _(src=src_lo):
            pltpu.sync_copy(src.at[pl.ds(slab0, NCH)], idx_all)

        @pl.when(c == 1)
        def _(src=src_hi):
            pltpu.sync_copy(src.at[pl.ds(slab0, NCH)], idx_all)

        pltpu.sync_copy(zeros, rows0)
        for k in range(SS):
            pltpu.sync_copy(rows0, acc_s.at[pl.ds(row0 + k * CH, CH)])
        plsc.subcore_barrier()

        gather_start(0, rows0, sg0)

        def body(k, carry):
            a = 2 * k
            gather_wait(rows0, sg0)
            scat_start(a, rows0, ss0, sd0, do_deg)

            @pl.when(k > 0)
            def _():
                scat_wait(rows1, ss1, sd1, do_deg)

            gather_start(a + 1, rows1, sg1)
            gather_wait(rows1, sg1)
            scat_start(a + 1, rows1, ss1, sd1, do_deg)
            scat_wait(rows0, ss0, sd0, do_deg)
            gather_start(a + 2, rows0, sg0)
            return carry

        lax.fori_loop(0, (NCH - 1) // 2, body, 0)

        gather_wait(rows0, sg0)
        scat_start(NCH - 1, rows0, ss0, sd0, do_deg)
        scat_wait(rows1, ss1, sd1, do_deg)
        scat_wait(rows0, ss0, sd0, do_deg)
        plsc.subcore_barrier()

        dst_lo = (out_q0, out_q1)[p]
        dst_hi = (out_q2, out_q3)[p]
        for k in range(SS):
            pltpu.sync_copy(acc_s.at[pl.ds(row0 + k * CH, CH)], rows0)

            @pl.when(c == 0)
            def _(dst=dst_lo, k=k):
                pltpu.sync_copy(rows0, dst.at[pl.ds(row0 + k * CH, CH)])

            @pl.when(c == 1)
            def _(dst=dst_hi, k=k):
                pltpu.sync_copy(rows0, dst.at[pl.ds(row0 + k * CH, CH)])

        if p == 0:
            @pl.when(c == 0)
            def _():
                pltpu.sync_copy(deg_s.at[pl.ds(row0, NRT)], dstage_v)
                pltpu.sync_copy(dstage_v, out_deg.at[pl.ds(row0, NRT)])


_sc_agg = functools.partial(
    pl.kernel,
    out_type=[
        jax.ShapeDtypeStruct((NP, DQ), jnp.float32),
        jax.ShapeDtypeStruct((NP, DQ), jnp.float32),
        jax.ShapeDtypeStruct((NP, DQ), jnp.float32),
        jax.ShapeDtypeStruct((NP, DQ), jnp.float32),
        jax.ShapeDtypeStruct((NP, DW), jnp.float32),
    ],
    mesh=plsc.VectorSubcoreMesh(core_axis_name="c", subcore_axis_name="s"),
    compiler_params=pltpu.CompilerParams(use_tc_tiling_on_sc=False),
    scratch_types=[
        pltpu.VMEM((NCH, CH), jnp.int32),
        pltpu.VMEM((NCH, CH), jnp.int32),
        pltpu.VMEM((CH, DQ), jnp.float32),
        pltpu.VMEM((CH, DQ), jnp.float32),
        pltpu.VMEM((CH, DW), jnp.float32),
        pltpu.VMEM((NRT, DW), jnp.float32),
        pltpu.VMEM_SHARED((NP, DQ), jnp.float32),
        pltpu.VMEM_SHARED((NP, DW), jnp.float32),
        pltpu.SemaphoreType.DMA,
        pltpu.SemaphoreType.DMA,
        pltpu.SemaphoreType.DMA,
        pltpu.SemaphoreType.DMA,
        pltpu.SemaphoreType.DMA,
        pltpu.SemaphoreType.DMA,
    ],
)(_sc_body)


BR = 1000


def _tc_body(feat_b, n0_b, n1_b, n2_b, n3_b, deg_b,
             ws, wn0, wn1, wn2, wn3, bias, out_b):
    inv = 1.0 / jnp.maximum(deg_b[...], 1.0)
    acc = jnp.dot(feat_b[...], ws[...], preferred_element_type=jnp.float32)
    acc += jnp.dot(n0_b[...] * inv, wn0[...], preferred_element_type=jnp.float32)
    acc += jnp.dot(n1_b[...] * inv, wn1[...], preferred_element_type=jnp.float32)
    acc += jnp.dot(n2_b[...] * inv, wn2[...], preferred_element_type=jnp.float32)
    acc += jnp.dot(n3_b[...] * inv, wn3[...], preferred_element_type=jnp.float32)
    out_b[...] = acc + bias[...]


def kernel(feat, edge_index, W_self, b_self, W_neigh, b_neigh):
    feat = feat.astype(jnp.float32)
    srcs = edge_index[0].astype(jnp.int32)
    dsts = edge_index[1].astype(jnp.int32)
    tbl = jnp.concatenate(
        [feat[:, 0 * DQ:1 * DQ], feat[:, 1 * DQ:2 * DQ],
         feat[:, 2 * DQ:3 * DQ], feat[:, 3 * DQ:4 * DQ]], axis=0)
    sq = [(srcs + q * N).reshape(NROWS, CH) for q in range(4)]
    d2d = dsts.reshape(NROWS, CH)
    zeros = jnp.zeros((CH, DQ), dtype=jnp.float32)
    zeros_d = jnp.zeros((NRT, DW), dtype=jnp.float32)

    n0, n1, n2, n3, deg16 = _sc_agg(tbl, sq[0], sq[1], sq[2], sq[3], d2d,
                                    zeros, zeros_d)
    deg = deg16[:N, :1]

    nblk = N // BR
    qspec = pl.BlockSpec((BR, DQ), lambda i: (i, 0))
    wspec = pl.BlockSpec((DQ, D), lambda i: (0, 0))
    out = pl.pallas_call(
        _tc_body,
        grid=(nblk,),
        in_specs=[
            pl.BlockSpec((BR, D), lambda i: (i, 0)),
            qspec, qspec, qspec, qspec,
            pl.BlockSpec((BR, 1), lambda i: (i, 0)),
            pl.BlockSpec((D, D), lambda i: (0, 0)),
            wspec, wspec, wspec, wspec,
            pl.BlockSpec((1, D), lambda i: (0, 0)),
        ],
        out_specs=pl.BlockSpec((BR, D), lambda i: (i, 0)),
        out_shape=jax.ShapeDtypeStruct((N, D), jnp.float32),
    )(feat, n0, n1, n2, n3, deg,
      W_self, W_neigh[0 * DQ:1 * DQ], W_neigh[1 * DQ:2 * DQ],
      W_neigh[2 * DQ:3 * DQ], W_neigh[3 * DQ:4 * DQ],
      (b_self + b_neigh).reshape(1, D))
    return out
